# Initial kernel scaffold; baseline (speedup 1.0000x reference)
import jax
import jax.numpy as jnp
from jax.experimental import pallas as pl


def kernel(x, emb, W_ih, W_hh, b_ih, b_hh, W1, b1, W2, b2, W3, b3, W4, b4, W5, b5):
    raise NotImplementedError("write your pallas kernel here")


# speedup vs baseline: 6.6651x; 6.6651x over previous
import functools

import jax
import jax.numpy as jnp
from jax import lax
from jax.experimental import pallas as pl
from jax.experimental.pallas import tpu as pltpu
from jax.experimental.pallas import tpu_sc as plsc

_H = 64
_CHUNK = 128


def _sc_gather_body(n_chunks, n_per_w, num_cores,
                    idx_hbm, emb_hbm, out_hbm, idx_v, rows_v, sem):
    wid = lax.axis_index("s") * num_cores + lax.axis_index("c")
    pltpu.sync_copy(idx_hbm.at[wid], idx_v)

    def fire(j, carry):
        dst = rows_v.at[pl.ds(j * _CHUNK, _CHUNK)]
        pltpu.async_copy(emb_hbm.at[idx_v.at[j]], dst, sem)
        return carry

    lax.fori_loop(0, n_chunks, fire, 0)
    pltpu.make_async_copy(emb_hbm.at[pl.ds(0, n_per_w)], rows_v, sem).wait()
    pltpu.sync_copy(rows_v, out_hbm.at[pl.ds(wid * n_per_w, n_per_w)])


@functools.lru_cache(maxsize=None)
def _make_sc_gather(V, D, N):
    info = plsc.get_sparse_core_info()
    nw = info.num_cores * info.num_subcores
    assert N % (nw * _CHUNK) == 0
    n_per_w = N // nw
    n_chunks = n_per_w // _CHUNK
    mesh = plsc.VectorSubcoreMesh(core_axis_name="c", subcore_axis_name="s")
    body = functools.partial(_sc_gather_body, n_chunks, n_per_w, info.num_cores)
    return nw, n_chunks, pl.kernel(
        body,
        out_type=jax.ShapeDtypeStruct((N, D), jnp.float32),
        mesh=mesh,
        scratch_types=[
            pltpu.VMEM((n_chunks, _CHUNK), jnp.int32),
            pltpu.VMEM((n_per_w, D), jnp.float32),
            pltpu.SemaphoreType.DMA,
        ],
        compiler_params=pltpu.CompilerParams(use_tc_tiling_on_sc=False),
    )


def _lstm_body(L, xs_ref, wih_ref, whh_ref, bg_ref,
               w1_ref, b1_ref, w2_ref, b2_ref, w3_ref, b3_ref,
               w4_ref, b4_ref, w5_ref, b5_ref, out_ref, h_ref, c_ref):
    l = pl.program_id(0)

    @pl.when(l == 0)
    def _init():
        h_ref[...] = jnp.zeros_like(h_ref)
        c_ref[...] = jnp.zeros_like(c_ref)

    xt = xs_ref[0]
    h = h_ref[...]
    g = (jnp.dot(xt, wih_ref[...], preferred_element_type=jnp.float32)
         + jnp.dot(h, whh_ref[...], preferred_element_type=jnp.float32)
         + bg_ref[...])
    i_g = jax.nn.sigmoid(g[:, 0 * _H:1 * _H])
    f_g = jax.nn.sigmoid(g[:, 1 * _H:2 * _H])
    g_g = jnp.tanh(g[:, 2 * _H:3 * _H])
    o_g = jax.nn.sigmoid(g[:, 3 * _H:4 * _H])
    c_new = f_g * c_ref[...] + i_g * g_g
    h_new = o_g * jnp.tanh(c_new)
    h_ref[...] = h_new
    c_ref[...] = c_new

    @pl.when(l == L - 1)
    def _head():
        o = jax.nn.relu(h_new)
        o = jax.nn.relu(
            jnp.dot(o, w1_ref[...], preferred_element_type=jnp.float32)
            + b1_ref[...])
        o = jax.nn.relu(
            jnp.dot(o, w2_ref[...], preferred_element_type=jnp.float32)
            + b2_ref[...])
        o = jax.nn.relu(
            jnp.dot(o, w3_ref[...], preferred_element_type=jnp.float32)
            + b3_ref[...])
        o = jax.nn.relu(
            jnp.dot(o, w4_ref[...], preferred_element_type=jnp.float32)
            + b4_ref[...])
        z = jnp.sum(o * w5_ref[...], axis=1, keepdims=True) + b5_ref[...]
        out_ref[...] = jax.nn.sigmoid(z)


@functools.lru_cache(maxsize=None)
def _make_lstm(L, B, D):
    full = lambda shape: pl.BlockSpec(shape, lambda l: (0,) * len(shape))
    return pl.pallas_call(
        functools.partial(_lstm_body, L),
        grid=(L,),
        in_specs=[
            pl.BlockSpec((1, B, D), lambda l: (l, 0, 0)),
            full((D, 4 * _H)),
            full((_H, 4 * _H)),
            full((1, 4 * _H)),
            full((_H, 64)),
            full((1, 64)),
            full((64, 128)),
            full((1, 128)),
            full((128, 64)),
            full((1, 64)),
            full((64, 32)),
            full((1, 32)),
            full((1, 32)),
            full((1, 1)),
        ],
        out_specs=pl.BlockSpec((B, 1), lambda l: (0, 0)),
        out_shape=jax.ShapeDtypeStruct((B, 1), jnp.float32),
        scratch_shapes=[
            pltpu.VMEM((B, _H), jnp.float32),
            pltpu.VMEM((B, _H), jnp.float32),
        ],
        compiler_params=pltpu.CompilerParams(
            dimension_semantics=("arbitrary",)),
    )


def kernel(x, emb, W_ih, W_hh, b_ih, b_hh,
           W1, b1, W2, b2, W3, b3, W4, b4, W5, b5):
    B, L = x.shape
    V, D = emb.shape
    N = B * L

    nw, n_chunks, gather = _make_sc_gather(V, D, N)
    idx = jnp.transpose(x).astype(jnp.int32).reshape(nw, n_chunks, _CHUNK)
    e_flat = gather(idx, emb)
    xs = e_flat.reshape(L, B, D)

    lstm = _make_lstm(L, B, D)
    return lstm(
        xs,
        jnp.transpose(W_ih), jnp.transpose(W_hh),
        (b_ih + b_hh).reshape(1, 4 * _H),
        jnp.transpose(W1), b1.reshape(1, -1),
        jnp.transpose(W2), b2.reshape(1, -1),
        jnp.transpose(W3), b3.reshape(1, -1),
        jnp.transpose(W4), b4.reshape(1, -1),
        W5.reshape(1, -1), b5.reshape(1, 1),
    )
